# Initial kernel scaffold; baseline (speedup 1.0000x reference)
import jax
import jax.numpy as jnp
from jax.experimental import pallas as pl


def kernel(token_inputs, W, b, num_experts, expert_capacity):
    raise NotImplementedError("write your pallas kernel here")


# speedup vs baseline: 2.1906x; 2.1906x over previous
import functools

import jax
import jax.numpy as jnp
from jax.experimental import pallas as pl
from jax.experimental.pallas import tpu as pltpu

_BLK = 256


def _router_body(x_ref, w_ref, b_ref, cap_ref, disp_ref, comb_ref, aux_ref, z_ref):
    g = pl.program_id(0)
    G = pl.num_programs(0)
    T = x_ref.shape[1]
    E = w_ref.shape[1]
    cap = cap_ref[0, 0]

    x = x_ref[0]
    w = w_ref[...]
    logits = jax.lax.dot_general(
        x, w, (((1,), (0,)), ((), ())),
        preferred_element_type=jnp.float32) + b_ref[...]

    m = jnp.max(logits, axis=-1, keepdims=True)
    ex = jnp.exp(logits - m)
    s = jnp.sum(ex, axis=-1, keepdims=True)
    probs = ex / s

    iota_e = jax.lax.broadcasted_iota(jnp.int32, (T, E), 1)
    v0 = jnp.max(probs, axis=-1, keepdims=True)
    e0 = jnp.min(jnp.where(probs == v0, iota_e, E), axis=-1, keepdims=True)
    probs1 = jnp.where(iota_e == e0, -jnp.inf, probs)
    v1 = jnp.max(probs1, axis=-1, keepdims=True)
    e1 = jnp.min(jnp.where(probs1 == v1, iota_e, E), axis=-1, keepdims=True)

    logz = m + jnp.log(s)
    z_part = jnp.sum(logz * logz) / (G * T)
    a0 = (iota_e == e0).astype(jnp.float32)
    a1 = (iota_e == e1).astype(jnp.float32)
    em_mean = jnp.sum(jnp.maximum(a0, a1), axis=0, keepdims=True) / T
    pm_mean = jnp.sum(probs, axis=0, keepdims=True) / T
    aux_part = jnp.sum(em_mean * pm_mean) * (E / G)

    @pl.when(g == 0)
    def _init():
        aux_ref[...] = jnp.zeros_like(aux_ref)
        z_ref[...] = jnp.zeros_like(z_ref)

    aux_ref[...] += jnp.reshape(aux_part, (1, 1))
    z_ref[...] += jnp.reshape(z_part, (1, 1))

    v0_row = jnp.transpose(v0)
    a_cat = jnp.concatenate([a0, a1], axis=1).astype(jnp.bfloat16)
    tot0 = jnp.sum(a0, axis=0, keepdims=True)

    iota_lane = jax.lax.broadcasted_iota(jnp.int32, (_BLK, T), 1)
    iota_sub = jax.lax.broadcasted_iota(jnp.int32, (_BLK, T), 0)

    for blk in range(T // _BLK):
        base = blk * _BLK
        vb = v0[base:base + _BLK, :]
        prec = (v0_row > vb) | ((v0_row == vb) & (iota_lane < iota_sub + base))
        counts = jax.lax.dot_general(
            prec.astype(jnp.bfloat16), a_cat, (((1,), (0,)), ((), ())),
            preferred_element_type=jnp.float32)
        a0b = a0[base:base + _BLK, :]
        a1b = a1[base:base + _BLK, :]
        p0 = jnp.sum(counts[:, :E] * a0b, axis=-1, keepdims=True)
        p1 = jnp.sum((counts[:, E:] + tot0) * a1b, axis=-1, keepdims=True)
        p0i = p0.astype(jnp.int32)
        p1i = p1.astype(jnp.int32)
        v0b = v0[base:base + _BLK, :]
        v1b = v1[base:base + _BLK, :]
        cw0 = jnp.where(p0i < cap, v0b, 0.0)
        cw1 = jnp.where(p1i < cap, v1b, 0.0)
        e0b = e0[base:base + _BLK, :]
        e1b = e1[base:base + _BLK, :]
        disp_ref[0, base:base + _BLK, :] = jnp.concatenate(
            [e0b, p0i, e1b, p1i], axis=1)
        comb_ref[0, base:base + _BLK, :] = jnp.concatenate([cw0, cw1], axis=1)


def _build_router(G, T, D, E, interpret=False):
    return pl.pallas_call(
        _router_body,
        grid=(G,),
        in_specs=[
            pl.BlockSpec((1, T, D), lambda g: (g, 0, 0)),
            pl.BlockSpec((D, E), lambda g: (0, 0)),
            pl.BlockSpec((1, E), lambda g: (0, 0)),
            pl.BlockSpec((1, 1), lambda g: (0, 0)),
        ],
        out_specs=[
            pl.BlockSpec((1, T, 4), lambda g: (g, 0, 0)),
            pl.BlockSpec((1, T, 2), lambda g: (g, 0, 0)),
            pl.BlockSpec((1, 1), lambda g: (0, 0)),
            pl.BlockSpec((1, 1), lambda g: (0, 0)),
        ],
        out_shape=[
            jax.ShapeDtypeStruct((G, T, 4), jnp.int32),
            jax.ShapeDtypeStruct((G, T, 2), jnp.float32),
            jax.ShapeDtypeStruct((1, 1), jnp.float32),
            jax.ShapeDtypeStruct((1, 1), jnp.float32),
        ],
        interpret=interpret,
    )


def _run(token_inputs, W, b, expert_capacity, interpret=False):
    G, T, D = token_inputs.shape
    E = W.shape[-1]
    cap = jnp.reshape(jnp.asarray(expert_capacity, jnp.int32), (1, 1))
    disp_raw, comb, aux, z = _build_router(G, T, D, E, interpret)(
        token_inputs, W, jnp.reshape(b, (1, E)), cap)
    disp = disp_raw.reshape(G, T, 2, 2)
    return disp, comb, aux[0, 0], z[0, 0]


def kernel(token_inputs, W, b, num_experts, expert_capacity):
    del num_experts
    return _run(token_inputs, W, b, expert_capacity)
